# baseline (speedup 1.0000x reference)
import functools

import jax
import jax.numpy as jnp
from jax import lax
from jax.experimental import pallas as pl
from jax.experimental.pallas import tpu as pltpu
from jax.experimental.pallas import tpu_sc as plsc

NUM_FEATS = 100000
EMB_DIM = 64
BATCH = 4096
N_FIELDS = 26
L = 16
NC, NS = 2, 16
NW = NC * NS
ROWS_PER_W = BATCH // NW
CB = 16
N_CHUNKS = ROWS_PER_W // CB
IDX_PER_CHUNK = CB * N_FIELDS
SUB = 4
IDX_PER_SUB = IDX_PER_CHUNK // SUB
KV = EMB_DIM // L


def _fm_body(xf, emb, bias, off, out, idx_v, rows_v, bias_v, out_v, off_v, sem):
    wid = lax.axis_index("s") * NC + lax.axis_index("c")
    pltpu.sync_copy(off, off_v)
    lane26 = lax.iota(jnp.int32, L) * N_FIELDS
    zero = jnp.zeros((L,), jnp.float32)

    def chunk_body(c, carry):
        row0 = wid * ROWS_PER_W + c * CB
        ib = row0 * N_FIELDS
        for j in range(SUB):
            pltpu.sync_copy(xf.at[pl.ds(ib + j * IDX_PER_SUB, IDX_PER_SUB)],
                            idx_v.at[j])
        def per_dim(d, tq):
            t, q = tq
            didx = jnp.full((L,), 0, jnp.int32) + d
            s = zero
            for f in range(N_FIELDS):
                v = plsc.load_gather(rows_v, [lane26 + f, didx])
                s = s + v
                q = q + v * v
            return (t + s * s, q)

        t, q = lax.fori_loop(0, EMB_DIM, per_dim, (zero, zero))
        bsum = zero
        for f in range(N_FIELDS):
            bsum = bsum + plsc.load_gather(bias_v, [lane26 + f])
        out_v[...] = 0.5 * (t - q) + bsum + off_v[...]
        pltpu.sync_copy(out_v, out.at[pl.ds(row0, CB)])
        return carry

    lax.fori_loop(0, N_CHUNKS, chunk_body, 0)


@functools.cache
def _fm_kernel():
    return functools.partial(
        pl.kernel,
        out_type=jax.ShapeDtypeStruct((BATCH,), jnp.float32),
        mesh=plsc.VectorSubcoreMesh(core_axis_name="c", subcore_axis_name="s"),
        compiler_params=pltpu.CompilerParams(
            needs_layout_passes=False, use_tc_tiling_on_sc=False),
        scratch_types=[
            pltpu.VMEM((SUB, IDX_PER_SUB), jnp.int32),
            pltpu.VMEM((IDX_PER_CHUNK, EMB_DIM), jnp.float32),
            pltpu.VMEM((IDX_PER_CHUNK,), jnp.float32),
            pltpu.VMEM((L,), jnp.float32),
            pltpu.VMEM((L,), jnp.float32),
            pltpu.SemaphoreType.DMA,
        ],
    )(_fm_body)


def kernel(X, x_emb_weight, x_bias, offset):
    xf = X.reshape(-1).astype(jnp.int32)
    off16 = jnp.broadcast_to(offset.astype(jnp.float32), (L,))
    return _fm_kernel()(xf, x_emb_weight, x_bias, off16)


# speedup vs baseline: 1.0142x; 1.0142x over previous
import functools

import jax
import jax.numpy as jnp
from jax import lax
from jax.experimental import pallas as pl
from jax.experimental.pallas import tpu as pltpu
from jax.experimental.pallas import tpu_sc as plsc

NUM_FEATS = 100000
EMB_DIM = 64
BATCH = 4096
N_FIELDS = 26
L = 16
NC, NS = 2, 16
NW = NC * NS
ROWS_PER_W = BATCH // NW
CB = 16
N_CHUNKS = ROWS_PER_W // CB
IDX_PER_W = ROWS_PER_W * N_FIELDS
IDX_PER_CHUNK = CB * N_FIELDS
SUB = 4
IDX_PER_SUB = IDX_PER_CHUNK // SUB
NBUF = 2


def _fm_body(xf, emb, bias, off, out, idx_v, rows_v, bias_v, out_v, off_v,
             sems):
    wid = lax.axis_index("s") * NC + lax.axis_index("c")
    pltpu.sync_copy(off, off_v)
    pltpu.sync_copy(xf.at[pl.ds(wid * IDX_PER_W, IDX_PER_W)], idx_v)

    lane = lax.iota(jnp.int32, L)
    lane_b = lane * N_FIELDS
    zero = jnp.zeros((L,), jnp.float32)
    def _rows_dst(buf, j):
        return rows_v.at[pl.ds(buf * IDX_PER_CHUNK + j * IDX_PER_SUB,
                               IDX_PER_SUB)]

    def fire(c, buf):
        for j in range(SUB):
            iv = idx_v.at[pl.ds(c * IDX_PER_CHUNK + j * IDX_PER_SUB,
                                IDX_PER_SUB)]
            pltpu.async_copy(emb.at[iv], _rows_dst(buf, j), sems.at[buf])
            pltpu.async_copy(
                bias.at[iv],
                bias_v.at[pl.ds(buf * IDX_PER_CHUNK + j * IDX_PER_SUB,
                                IDX_PER_SUB)],
                sems.at[buf])

    def drain(c, buf):
        for j in range(SUB):
            pltpu.make_async_copy(
                emb.at[idx_v.at[pl.ds(0, IDX_PER_SUB)]],
                _rows_dst(buf, j), sems.at[buf]).wait()
            pltpu.make_async_copy(
                bias.at[idx_v.at[pl.ds(0, IDX_PER_SUB)]],
                bias_v.at[pl.ds(buf * IDX_PER_CHUNK + j * IDX_PER_SUB,
                                IDX_PER_SUB)],
                sems.at[buf]).wait()

    fire(0, 0)
    for c in range(N_CHUNKS):
        buf = c % NBUF
        drain(c, buf)
        if c + 1 < N_CHUNKS:
            fire(c + 1, (c + 1) % NBUF)
        rbase = buf * IDX_PER_CHUNK + lane_b
        bbase = buf * IDX_PER_CHUNK + lane_b

        def per_dim(d, tq):
            t, q0, q1 = tq
            didx = jnp.full((L,), 0, jnp.int32) + d
            sa, sb = zero, zero
            qa, qb = zero, zero
            for f in range(N_FIELDS):
                v = plsc.load_gather(rows_v, [rbase + f, didx])
                if f % 2 == 0:
                    sa = sa + v
                    qa = qa + v * v
                else:
                    sb = sb + v
                    qb = qb + v * v
            s = sa + sb
            return (t + s * s, q0 + qa, q1 + qb)

        t, q0, q1 = lax.fori_loop(0, EMB_DIM, per_dim, (zero, zero, zero))
        bsum = zero
        for f in range(N_FIELDS):
            bsum = bsum + plsc.load_gather(bias_v, [bbase + f])
        out_v[...] = 0.5 * (t - q0 - q1) + bsum + off_v[...]
        pltpu.sync_copy(out_v,
                        out.at[pl.ds(wid * ROWS_PER_W + c * CB, CB)])


@functools.cache
def _fm_kernel():
    return functools.partial(
        pl.kernel,
        out_type=jax.ShapeDtypeStruct((BATCH,), jnp.float32),
        mesh=plsc.VectorSubcoreMesh(core_axis_name="c", subcore_axis_name="s"),
        compiler_params=pltpu.CompilerParams(
            needs_layout_passes=False, use_tc_tiling_on_sc=False),
        scratch_types=[
            pltpu.VMEM((IDX_PER_W,), jnp.int32),
            pltpu.VMEM((NBUF * IDX_PER_CHUNK, EMB_DIM), jnp.float32),
            pltpu.VMEM((NBUF * IDX_PER_CHUNK,), jnp.float32),
            pltpu.VMEM((L,), jnp.float32),
            pltpu.VMEM((L,), jnp.float32),
            pltpu.SemaphoreType.DMA((NBUF,)),
        ],
    )(_fm_body)


def kernel(X, x_emb_weight, x_bias, offset):
    xf = X.reshape(-1).astype(jnp.int32)
    off16 = jnp.broadcast_to(offset.astype(jnp.float32), (L,))
    return _fm_kernel()(xf, x_emb_weight, x_bias, off16)


# speedup vs baseline: 2.0207x; 1.9923x over previous
import functools

import jax
import jax.numpy as jnp
from jax import lax
from jax.experimental import pallas as pl
from jax.experimental.pallas import tpu as pltpu
from jax.experimental.pallas import tpu_sc as plsc

NUM_FEATS = 100000
EMB_DIM = 64
BATCH = 4096
N_FIELDS = 26
L = 16
NC, NS = 2, 16
NW = NC * NS
ROWS_PER_W = BATCH // NW
CB = 16
N_CHUNKS = ROWS_PER_W // CB
IDX_PER_W = ROWS_PER_W * N_FIELDS
IDX_PER_CHUNK = CB * N_FIELDS
SUB = 4
IDX_PER_SUB = IDX_PER_CHUNK // SUB
NBUF = 2
DG = 4
N_GROUPS = EMB_DIM // DG


def _fm_body(xf, emb, bias, off, out, idx_v, rows_v, bias_v, out_v, off_v,
             sems):
    wid = lax.axis_index("s") * NC + lax.axis_index("c")
    pltpu.sync_copy(off, off_v)
    pltpu.sync_copy(xf.at[pl.ds(wid * IDX_PER_W, IDX_PER_W)], idx_v)

    lane = lax.iota(jnp.int32, L)
    lane_b = lane * N_FIELDS
    zero = jnp.zeros((L,), jnp.float32)

    def fire(c, buf):
        for j in range(SUB):
            iv = idx_v.at[pl.ds(c * IDX_PER_CHUNK + j * IDX_PER_SUB,
                                IDX_PER_SUB)]
            pltpu.async_copy(
                emb.at[iv],
                rows_v.at[pl.ds(buf * IDX_PER_CHUNK + j * IDX_PER_SUB,
                                IDX_PER_SUB)],
                sems.at[buf])
            pltpu.async_copy(
                bias.at[iv],
                bias_v.at[pl.ds(buf * IDX_PER_CHUNK + j * IDX_PER_SUB,
                                IDX_PER_SUB)],
                sems.at[buf])

    def drain(buf):
        for j in range(SUB):
            pltpu.make_async_copy(
                emb.at[idx_v.at[pl.ds(0, IDX_PER_SUB)]],
                rows_v.at[pl.ds(buf * IDX_PER_CHUNK + j * IDX_PER_SUB,
                                IDX_PER_SUB)],
                sems.at[buf]).wait()
            pltpu.make_async_copy(
                bias.at[idx_v.at[pl.ds(0, IDX_PER_SUB)]],
                bias_v.at[pl.ds(buf * IDX_PER_CHUNK + j * IDX_PER_SUB,
                                IDX_PER_SUB)],
                sems.at[buf]).wait()

    def compute(c, buf):
        rbase = buf * IDX_PER_CHUNK + lane_b

        def per_group(g, carry):
            t, qt = carry
            dvec = lane + g * DG
            didx = [(dvec + dd) & (EMB_DIM - 1) for dd in range(DG)]
            s = [zero] * DG
            q = [zero] * DG
            for f in range(N_FIELDS):
                ridx = rbase + f
                for dd in range(DG):
                    v = plsc.load_gather(rows_v, [ridx, didx[dd]])
                    s[dd] = s[dd] + v
                    q[dd] = q[dd] + v * v
            for dd in range(DG):
                t = t + s[dd] * s[dd]
                qt = qt + q[dd]
            return (t, qt)

        t, qt = lax.fori_loop(0, N_GROUPS, per_group, (zero, zero))
        bsum = zero
        for f in range(N_FIELDS):
            bsum = bsum + plsc.load_gather(bias_v,
                                           [buf * IDX_PER_CHUNK + lane_b + f])
        out_v[...] = 0.5 * (t - qt) + bsum + off_v[...]
        pltpu.sync_copy(out_v, out.at[pl.ds(wid * ROWS_PER_W + c * CB, CB)])

    fire(0, 0)

    def super_body(i, carry):
        c0 = 2 * i
        drain(0)
        fire(c0 + 1, 1)
        compute(c0, 0)
        drain(1)

        @pl.when(i < N_CHUNKS // 2 - 1)
        def _():
            fire(c0 + 2, 0)

        compute(c0 + 1, 1)
        return carry

    lax.fori_loop(0, N_CHUNKS // 2, super_body, 0)


@functools.cache
def _fm_kernel():
    return functools.partial(
        pl.kernel,
        out_type=jax.ShapeDtypeStruct((BATCH,), jnp.float32),
        mesh=plsc.VectorSubcoreMesh(core_axis_name="c", subcore_axis_name="s"),
        compiler_params=pltpu.CompilerParams(
            needs_layout_passes=False, use_tc_tiling_on_sc=False),
        scratch_types=[
            pltpu.VMEM((IDX_PER_W,), jnp.int32),
            pltpu.VMEM((NBUF * IDX_PER_CHUNK, EMB_DIM), jnp.float32),
            pltpu.VMEM((NBUF * IDX_PER_CHUNK,), jnp.float32),
            pltpu.VMEM((L,), jnp.float32),
            pltpu.VMEM((L,), jnp.float32),
            pltpu.SemaphoreType.DMA((NBUF,)),
        ],
    )(_fm_body)


def kernel(X, x_emb_weight, x_bias, offset):
    xf = X.reshape(-1).astype(jnp.int32)
    off16 = jnp.broadcast_to(offset.astype(jnp.float32), (L,))
    return _fm_kernel()(xf, x_emb_weight, x_bias, off16)
